# Initial kernel scaffold; baseline (speedup 1.0000x reference)
import jax
import jax.numpy as jnp
from jax.experimental import pallas as pl


def kernel(x, edge_index, W1, b1, W2, b2, W3, b3):
    raise NotImplementedError("write your pallas kernel here")


# speedup vs baseline: 18.8693x; 18.8693x over previous
import functools

import jax
import jax.numpy as jnp
from jax import lax
from jax.experimental import pallas as pl
from jax.experimental.pallas import tpu as pltpu
from jax.experimental.pallas import tpu_sc as plsc

N = 10000
NP = 10240
E = 320000
D_IN = 128
H = 64
D_OUT = 128

NC = 2
NS = 16
NW = NC * NS
EPW = E // NW
CH = 80
NCHUNK = EPW // CH
RPS = NP // NS

_mesh = plsc.VectorSubcoreMesh(core_axis_name="c", subcore_axis_name="s")
_sc_params = pltpu.CompilerParams(use_tc_tiling_on_sc=False)


def _deg_body(dst_hbm, ones_hbm, zeros_hbm, out_hbm, idx_v, ones_v, acc):
    cid = lax.axis_index("c")
    sid = lax.axis_index("s")
    wid = sid * NC + cid
    my = pl.ds(sid * RPS, RPS)
    pltpu.sync_copy(zeros_hbm.at[my], acc.at[my])
    pltpu.sync_copy(ones_hbm, ones_v)
    pltpu.sync_copy(dst_hbm.at[wid], idx_v)
    plsc.subcore_barrier()

    @pl.loop(0, NCHUNK)
    def _(ci):
        pltpu.sync_copy(ones_v, acc.at[idx_v.at[ci]], add=True)

    plsc.subcore_barrier()
    pltpu.sync_copy(acc.at[my], out_hbm.at[cid, my])


@jax.jit
def _deg_partials(dst3, ones16, zeros16):
    k = pl.kernel(
        _deg_body,
        out_type=jax.ShapeDtypeStruct((NC, NP, 16), jnp.float32),
        mesh=_mesh,
        scratch_types=[
            pltpu.VMEM((NCHUNK, CH), jnp.int32),
            pltpu.VMEM((CH, 16), jnp.float32),
            pltpu.VMEM_SHARED((NP, 16), jnp.float32),
        ],
        compiler_params=_sc_params,
    )
    return k(dst3, ones16, zeros16)


def _mp_body(g_hbm, src_hbm, dst_hbm, zeros_hbm, out_hbm,
             sidx_v, didx_v, rows_v, acc, sem):
    cid = lax.axis_index("c")
    sid = lax.axis_index("s")
    wid = sid * NC + cid
    my = pl.ds(sid * RPS, RPS)
    pltpu.sync_copy(zeros_hbm.at[my], acc.at[my])
    pltpu.sync_copy(src_hbm.at[wid], sidx_v)
    pltpu.sync_copy(dst_hbm.at[wid], didx_v)
    plsc.subcore_barrier()

    @pl.loop(0, NCHUNK)
    def _(ci):
        pltpu.async_copy(g_hbm.at[sidx_v.at[ci]], rows_v, sem).wait()
        pltpu.sync_copy(rows_v, acc.at[didx_v.at[ci]], add=True)

    plsc.subcore_barrier()
    pltpu.sync_copy(acc.at[my], out_hbm.at[cid, my])


def _make_mp(h):
    @jax.jit
    def mp(g, src3, dst3, zeros):
        k = pl.kernel(
            _mp_body,
            out_type=jax.ShapeDtypeStruct((NC, NP, h), jnp.float32),
            mesh=_mesh,
            scratch_types=[
                pltpu.VMEM((NCHUNK, CH), jnp.int32),
                pltpu.VMEM((NCHUNK, CH), jnp.int32),
                pltpu.VMEM((CH, h), jnp.float32),
                pltpu.VMEM_SHARED((NP, h), jnp.float32),
                pltpu.SemaphoreType.DMA,
            ],
            compiler_params=_sc_params,
        )
        return k(g, src3, dst3, zeros)
    return mp


_mp64 = _make_mp(H)
_mp128 = _make_mp(D_OUT)


_R = 1024


def _tc1_body(p_ref, x_ref, w_ref, g_ref, dis_ref):
    p = p_ref[...]
    deg = p[0, :, 0:1] + p[1, :, 0:1] + 1.0
    dis = lax.rsqrt(deg)
    g_ref[...] = jnp.dot(x_ref[...], w_ref[...],
                         preferred_element_type=jnp.float32) * dis
    dis_ref[...] = dis


@jax.jit
def _tc1(degp, x, W1):
    return pl.pallas_call(
        _tc1_body,
        grid=(NP // _R,),
        in_specs=[
            pl.BlockSpec((NC, _R, 16), lambda i: (0, i, 0)),
            pl.BlockSpec((_R, D_IN), lambda i: (i, 0)),
            pl.BlockSpec((D_IN, H), lambda i: (0, 0)),
        ],
        out_specs=[
            pl.BlockSpec((_R, H), lambda i: (i, 0)),
            pl.BlockSpec((_R, 1), lambda i: (i, 0)),
        ],
        out_shape=[
            jax.ShapeDtypeStruct((NP, H), jnp.float32),
            jax.ShapeDtypeStruct((NP, 1), jnp.float32),
        ],
    )(degp, x, W1)


def _tc_mid_body(p_ref, g_ref, dis_ref, b_ref, w_ref, out_ref):
    p = p_ref[...]
    dis = dis_ref[...]
    h = jnp.maximum(dis * (p[0] + p[1] + g_ref[...]) + b_ref[...], 0.0)
    out_ref[...] = jnp.dot(h, w_ref[...],
                           preferred_element_type=jnp.float32) * dis


def _make_tc_mid(h_in, h_out):
    @jax.jit
    def tc_mid(partials, g, dis, b2d, W):
        return pl.pallas_call(
            _tc_mid_body,
            grid=(NP // _R,),
            in_specs=[
                pl.BlockSpec((NC, _R, h_in), lambda i: (0, i, 0)),
                pl.BlockSpec((_R, h_in), lambda i: (i, 0)),
                pl.BlockSpec((_R, 1), lambda i: (i, 0)),
                pl.BlockSpec((1, h_in), lambda i: (0, 0)),
                pl.BlockSpec((h_in, h_out), lambda i: (0, 0)),
            ],
            out_specs=pl.BlockSpec((_R, h_out), lambda i: (i, 0)),
            out_shape=jax.ShapeDtypeStruct((NP, h_out), jnp.float32),
        )(partials, g, dis, b2d, W)
    return tc_mid


_tc2 = _make_tc_mid(H, H)
_tc3 = _make_tc_mid(H, D_OUT)


def _tc4_body(p_ref, g_ref, dis_ref, b_ref, out_ref):
    p = p_ref[...]
    out_ref[...] = dis_ref[...] * (p[0] + p[1] + g_ref[...]) + b_ref[...]


@jax.jit
def _tc4(partials, g, dis, b2d):
    return pl.pallas_call(
        _tc4_body,
        grid=(NP // _R,),
        in_specs=[
            pl.BlockSpec((NC, _R, D_OUT), lambda i: (0, i, 0)),
            pl.BlockSpec((_R, D_OUT), lambda i: (i, 0)),
            pl.BlockSpec((_R, 1), lambda i: (i, 0)),
            pl.BlockSpec((1, D_OUT), lambda i: (0, 0)),
        ],
        out_specs=pl.BlockSpec((_R, D_OUT), lambda i: (i, 0)),
        out_shape=jax.ShapeDtypeStruct((NP, D_OUT), jnp.float32),
    )(partials, g, dis, b2d)


def kernel(x, edge_index, W1, b1, W2, b2, W3, b3):
    src3 = edge_index[0].reshape(NW, NCHUNK, CH)
    dst3 = edge_index[1].reshape(NW, NCHUNK, CH)
    ones16 = jnp.ones((CH, 16), jnp.float32)
    z16 = jnp.zeros((NP, 16), jnp.float32)
    z64 = jnp.zeros((NP, H), jnp.float32)
    z128 = jnp.zeros((NP, D_OUT), jnp.float32)
    xp = jnp.pad(x, ((0, NP - N), (0, 0)))

    degp = _deg_partials(dst3, ones16, z16)
    g1, dis = _tc1(degp, xp, W1)
    p1 = _mp64(g1, src3, dst3, z64)
    g2 = _tc2(p1, g1, dis, b1.reshape(1, H), W2)
    p2 = _mp64(g2, src3, dst3, z64)
    g3 = _tc3(p2, g2, dis, b2.reshape(1, H), W3)
    p3 = _mp128(g3, src3, dst3, z128)
    out = _tc4(p3, g3, dis, b3.reshape(1, D_OUT))
    return out[:N]


# speedup vs baseline: 28.3780x; 1.5039x over previous
import functools

import jax
import jax.numpy as jnp
from jax import lax
from jax.experimental import pallas as pl
from jax.experimental.pallas import tpu as pltpu
from jax.experimental.pallas import tpu_sc as plsc

N = 10000
NP = 10240
E = 320000
D_IN = 128
H = 64
D_OUT = 128

NC = 2
NS = 16
NW = NC * NS
EPW = E // NW
CH = 80
NCHUNK = EPW // CH
RPS = NP // NS

_mesh = plsc.VectorSubcoreMesh(core_axis_name="c", subcore_axis_name="s")
_sc_params = pltpu.CompilerParams(use_tc_tiling_on_sc=False)


def _deg_body(dst_hbm, ones_hbm, zeros_hbm, out_hbm, idx_v, ones_v, acc):
    cid = lax.axis_index("c")
    sid = lax.axis_index("s")
    wid = sid * NC + cid
    my = pl.ds(sid * RPS, RPS)
    pltpu.sync_copy(zeros_hbm.at[my], acc.at[my])
    pltpu.sync_copy(ones_hbm, ones_v)
    pltpu.sync_copy(dst_hbm.at[wid], idx_v)
    plsc.subcore_barrier()

    @pl.loop(0, NCHUNK)
    def _(ci):
        pltpu.sync_copy(ones_v, acc.at[idx_v.at[ci]], add=True)

    plsc.subcore_barrier()
    pltpu.sync_copy(acc.at[my], out_hbm.at[cid, my])


@jax.jit
def _deg_partials(dst3, ones16, zeros16):
    k = pl.kernel(
        _deg_body,
        out_type=jax.ShapeDtypeStruct((NC, NP, 16), jnp.float32),
        mesh=_mesh,
        scratch_types=[
            pltpu.VMEM((NCHUNK, CH), jnp.int32),
            pltpu.VMEM((CH, 16), jnp.float32),
            pltpu.VMEM_SHARED((NP, 16), jnp.float32),
        ],
        compiler_params=_sc_params,
    )
    return k(dst3, ones16, zeros16)


def _mp_body(g_hbm, src_hbm, dst_hbm, zeros_hbm, out_hbm,
             sidx_v, didx_v, rows_a, rows_b, acc, sem_a, sem_b):
    cid = lax.axis_index("c")
    sid = lax.axis_index("s")
    wid = sid * NC + cid
    my = pl.ds(sid * RPS, RPS)
    pltpu.sync_copy(zeros_hbm.at[my], acc.at[my])
    pltpu.sync_copy(src_hbm.at[wid], sidx_v)
    pltpu.sync_copy(dst_hbm.at[wid], didx_v)
    plsc.subcore_barrier()

    pltpu.async_copy(g_hbm.at[sidx_v.at[0]], rows_a, sem_a)

    @pl.loop(0, NCHUNK - 1, step=2)
    def _(c):
        pltpu.async_copy(g_hbm.at[sidx_v.at[c + 1]], rows_b, sem_b)
        pltpu.make_async_copy(g_hbm.at[sidx_v.at[c]], rows_a, sem_a).wait()
        pltpu.sync_copy(rows_a, acc.at[didx_v.at[c]], add=True)
        pltpu.async_copy(g_hbm.at[sidx_v.at[c + 2]], rows_a, sem_a)
        pltpu.make_async_copy(g_hbm.at[sidx_v.at[c + 1]], rows_b, sem_b).wait()
        pltpu.sync_copy(rows_b, acc.at[didx_v.at[c + 1]], add=True)

    pltpu.make_async_copy(g_hbm.at[sidx_v.at[NCHUNK - 1]], rows_a, sem_a).wait()
    pltpu.sync_copy(rows_a, acc.at[didx_v.at[NCHUNK - 1]], add=True)

    plsc.subcore_barrier()
    pltpu.sync_copy(acc.at[my], out_hbm.at[cid, my])


def _make_mp(h):
    @jax.jit
    def mp(g, src3, dst3, zeros):
        k = pl.kernel(
            _mp_body,
            out_type=jax.ShapeDtypeStruct((NC, NP, h), jnp.float32),
            mesh=_mesh,
            scratch_types=[
                pltpu.VMEM((NCHUNK, CH), jnp.int32),
                pltpu.VMEM((NCHUNK, CH), jnp.int32),
                pltpu.VMEM((CH, h), jnp.float32),
                pltpu.VMEM((CH, h), jnp.float32),
                pltpu.VMEM_SHARED((NP, h), jnp.float32),
                pltpu.SemaphoreType.DMA,
                pltpu.SemaphoreType.DMA,
            ],
            compiler_params=_sc_params,
        )
        return k(g, src3, dst3, zeros)
    return mp


_mp64 = _make_mp(H)
_mp128 = _make_mp(D_OUT)


_R = 1024


def _tc1_body(p_ref, x_ref, w_ref, g_ref, dis_ref):
    p = p_ref[...]
    deg = p[0, :, 0:1] + p[1, :, 0:1] + 1.0
    dis = lax.rsqrt(deg)
    g_ref[...] = jnp.dot(x_ref[...], w_ref[...],
                         preferred_element_type=jnp.float32) * dis
    dis_ref[...] = dis


@jax.jit
def _tc1(degp, x, W1):
    return pl.pallas_call(
        _tc1_body,
        grid=(NP // _R,),
        in_specs=[
            pl.BlockSpec((NC, _R, 16), lambda i: (0, i, 0)),
            pl.BlockSpec((_R, D_IN), lambda i: (i, 0)),
            pl.BlockSpec((D_IN, H), lambda i: (0, 0)),
        ],
        out_specs=[
            pl.BlockSpec((_R, H), lambda i: (i, 0)),
            pl.BlockSpec((_R, 1), lambda i: (i, 0)),
        ],
        out_shape=[
            jax.ShapeDtypeStruct((NP, H), jnp.float32),
            jax.ShapeDtypeStruct((NP, 1), jnp.float32),
        ],
    )(degp, x, W1)


def _tc_mid_body(p_ref, g_ref, dis_ref, b_ref, w_ref, out_ref):
    p = p_ref[...]
    dis = dis_ref[...]
    h = jnp.maximum(dis * (p[0] + p[1] + g_ref[...]) + b_ref[...], 0.0)
    out_ref[...] = jnp.dot(h, w_ref[...],
                           preferred_element_type=jnp.float32) * dis


def _make_tc_mid(h_in, h_out):
    @jax.jit
    def tc_mid(partials, g, dis, b2d, W):
        return pl.pallas_call(
            _tc_mid_body,
            grid=(NP // _R,),
            in_specs=[
                pl.BlockSpec((NC, _R, h_in), lambda i: (0, i, 0)),
                pl.BlockSpec((_R, h_in), lambda i: (i, 0)),
                pl.BlockSpec((_R, 1), lambda i: (i, 0)),
                pl.BlockSpec((1, h_in), lambda i: (0, 0)),
                pl.BlockSpec((h_in, h_out), lambda i: (0, 0)),
            ],
            out_specs=pl.BlockSpec((_R, h_out), lambda i: (i, 0)),
            out_shape=jax.ShapeDtypeStruct((NP, h_out), jnp.float32),
        )(partials, g, dis, b2d, W)
    return tc_mid


_tc2 = _make_tc_mid(H, H)
_tc3 = _make_tc_mid(H, D_OUT)


def _tc4_body(p_ref, g_ref, dis_ref, b_ref, out_ref):
    p = p_ref[...]
    out_ref[...] = dis_ref[...] * (p[0] + p[1] + g_ref[...]) + b_ref[...]


@jax.jit
def _tc4(partials, g, dis, b2d):
    return pl.pallas_call(
        _tc4_body,
        grid=(NP // _R,),
        in_specs=[
            pl.BlockSpec((NC, _R, D_OUT), lambda i: (0, i, 0)),
            pl.BlockSpec((_R, D_OUT), lambda i: (i, 0)),
            pl.BlockSpec((_R, 1), lambda i: (i, 0)),
            pl.BlockSpec((1, D_OUT), lambda i: (0, 0)),
        ],
        out_specs=pl.BlockSpec((_R, D_OUT), lambda i: (i, 0)),
        out_shape=jax.ShapeDtypeStruct((NP, D_OUT), jnp.float32),
    )(partials, g, dis, b2d)


def kernel(x, edge_index, W1, b1, W2, b2, W3, b3):
    src3 = edge_index[0].reshape(NW, NCHUNK, CH)
    dst3 = edge_index[1].reshape(NW, NCHUNK, CH)
    ones16 = jnp.ones((CH, 16), jnp.float32)
    z16 = jnp.zeros((NP, 16), jnp.float32)
    z64 = jnp.zeros((NP, H), jnp.float32)
    z128 = jnp.zeros((NP, D_OUT), jnp.float32)
    xp = jnp.pad(x, ((0, NP - N), (0, 0)))

    degp = _deg_partials(dst3, ones16, z16)
    g1, dis = _tc1(degp, xp, W1)
    p1 = _mp64(g1, src3, dst3, z64)
    g2 = _tc2(p1, g1, dis, b1.reshape(1, H), W2)
    p2 = _mp64(g2, src3, dst3, z64)
    g3 = _tc3(p2, g2, dis, b2.reshape(1, H), W3)
    p3 = _mp128(g3, src3, dst3, z128)
    out = _tc4(p3, g3, dis, b3.reshape(1, D_OUT))
    return out[:N]
